# baseline (speedup 1.0000x reference)
import functools

import jax
import jax.numpy as jnp
from jax import lax
from jax.experimental import pallas as pl
from jax.experimental.pallas import tpu as pltpu
from jax.experimental.pallas import tpu_sc as plsc

_N = 10000
_DEG = 32
_D = 128
_UNITS = 128
_LANES = 16
_NC = 2
_NS = 16
_NW = _NC * _NS
_CHUNK = 8

_BN = 400
_N_SC = 8000
_N_TC = _N - _N_SC


def _oems_pairs(n):
    pairs = []

    def _sort(lo, m):
        if m > 1:
            k = m // 2
            _sort(lo, k)
            _sort(lo + k, k)
            _merge(lo, m, 1)

    def _merge(lo, m, r):
        step = r * 2
        if step < m:
            _merge(lo, m, step)
            _merge(lo + r, m, step)
            for i in range(lo + r, lo + m - r, step):
                pairs.append((i, i + r))
        else:
            pairs.append((lo, lo + r))

    _sort(0, n)
    return pairs


_PAIRS16 = _oems_pairs(16)


def _median32(vals):
    a = list(vals[:16])
    b = list(vals[16:])
    for i, j in _PAIRS16:
        lo = jnp.minimum(a[i], a[j])
        a[j] = jnp.maximum(a[i], a[j])
        a[i] = lo
        lo = jnp.minimum(b[i], b[j])
        b[j] = jnp.maximum(b[i], b[j])
        b[i] = lo
    cand = [jnp.maximum(a[i], b[15 - i]) for i in range(16)]
    while len(cand) > 1:
        cand = [jnp.minimum(cand[2 * i], cand[2 * i + 1])
                for i in range(len(cand) // 2)]
    return cand[0]


_NCHUNKS = _N_SC // _CHUNK
_MAX_NK = (_NCHUNKS + _NW - 1) // _NW
_NPAIRS = (_MAX_NK + 1) // 2


def _sc_median_body(neigh_hbm, med_hbm, buf0, buf1, obuf0, obuf1,
                    isem0, isem1, osem0, osem1):
    wid = lax.axis_index("s") * _NC + lax.axis_index("c")
    nk = (_NCHUNKS - wid + (_NW - 1)) // _NW
    bufs = (buf0, buf1)
    obufs = (obuf0, obuf1)
    isems = (isem0, isem1)
    osems = (osem0, osem1)

    def _in_copy(k, b):
        base = (wid + k * _NW) * _CHUNK
        return pltpu.make_async_copy(
            neigh_hbm.at[pl.ds(base, _CHUNK)], bufs[b], isems[b])

    def _out_copy(k, b):
        base = (wid + k * _NW) * _CHUNK
        return pltpu.make_async_copy(
            obufs[b], med_hbm.at[pl.ds(base, _CHUNK)], osems[b])

    _in_copy(0, 0).start()
    _in_copy(1, 1).start()

    def _compute(b):
        buf = bufs[b]
        obuf = obufs[b]

        def node_body(nd, c):
            for fg in range(_D // _LANES):
                col = fg * _LANES
                vals = [buf[nd, s, pl.ds(col, _LANES)] for s in range(_DEG)]
                obuf[nd, pl.ds(col, _LANES)] = _median32(vals)
            return c

        lax.fori_loop(0, _CHUNK, node_body, 0)

    def pair_body(p, c):
        for b in range(2):
            k = 2 * p + b

            def slot(k=k, b=b):
                _in_copy(k, b).wait()
                pl.when(k >= 2)(lambda: _out_copy(k - 2, b).wait())
                _compute(b)
                _out_copy(k, b).start()
                pl.when(k + 2 < nk)(lambda: _in_copy(k + 2, b).start())

            pl.when(k < nk)(slot)
        return c

    lax.fori_loop(0, _NPAIRS, pair_body, 0)
    _out_copy(nk - 2 + (nk % 2), 0).wait()
    _out_copy(nk - 1 - (nk % 2), 1).wait()


@functools.cache
def _sc_median_kernel():
    return pl.kernel(
        _sc_median_body,
        out_type=jax.ShapeDtypeStruct((_N_SC, _D), jnp.float32),
        mesh=plsc.VectorSubcoreMesh(core_axis_name="c", subcore_axis_name="s",
                                    num_cores=_NC, num_subcores=_NS),
        scratch_types=[
            pltpu.VMEM((_CHUNK, _DEG, _D), jnp.float32),
            pltpu.VMEM((_CHUNK, _DEG, _D), jnp.float32),
            pltpu.VMEM((_CHUNK, _D), jnp.float32),
            pltpu.VMEM((_CHUNK, _D), jnp.float32),
            pltpu.SemaphoreType.DMA,
            pltpu.SemaphoreType.DMA,
            pltpu.SemaphoreType.DMA,
            pltpu.SemaphoreType.DMA,
        ],
    )


_BNT = 16


def _tc_fused_body(x_ref, neigh_ref, ws_ref, wn_ref, b_ref, o_ref, med_ref):
    for r in range(_BNT // 8):
        rows = pl.ds(r * 8, 8)
        med_ref[rows, :] = _median32(
            [neigh_ref[rows, s, :] for s in range(_DEG)])
    out = jnp.dot(x_ref[...], ws_ref[...], preferred_element_type=jnp.float32)
    out = out + jnp.dot(med_ref[...], wn_ref[...],
                        preferred_element_type=jnp.float32)
    o_ref[...] = jnp.maximum(out + b_ref[...], 0.0)


def _tc_fused(x, neigh_x, ws, wn, bias2):
    off = _N_SC // _BNT
    return pl.pallas_call(
        _tc_fused_body,
        grid=(_N_TC // _BNT,),
        in_specs=[
            pl.BlockSpec((_BNT, _D), lambda i: (off + i, 0)),
            pl.BlockSpec((_BNT, _DEG, _D), lambda i: (off + i, 0, 0)),
            pl.BlockSpec((_D, _UNITS), lambda i: (0, 0)),
            pl.BlockSpec((_D, _UNITS), lambda i: (0, 0)),
            pl.BlockSpec((1, _UNITS), lambda i: (0, 0)),
        ],
        out_specs=pl.BlockSpec((_BNT, _UNITS), lambda i: (i, 0)),
        out_shape=jax.ShapeDtypeStruct((_N_TC, _UNITS), jnp.float32),
        scratch_shapes=[pltpu.VMEM((_BNT, _D), jnp.float32)],
    )(x, neigh_x, ws, wn, bias2)


def _tc_matmul_body(x_ref, med_ref, ws_ref, wn_ref, b_ref, o_ref):
    acc = jnp.dot(x_ref[...], ws_ref[...], preferred_element_type=jnp.float32)
    acc = acc + jnp.dot(med_ref[...], wn_ref[...],
                        preferred_element_type=jnp.float32)
    o_ref[...] = jnp.maximum(acc + b_ref[...], 0.0)


def _tc_matmul(x, med, ws, wn, bias2):
    return pl.pallas_call(
        _tc_matmul_body,
        grid=(_N_SC // _BN,),
        in_specs=[
            pl.BlockSpec((_BN, _D), lambda i: (i, 0)),
            pl.BlockSpec((_BN, _D), lambda i: (i, 0)),
            pl.BlockSpec((_D, _UNITS), lambda i: (0, 0)),
            pl.BlockSpec((_D, _UNITS), lambda i: (0, 0)),
            pl.BlockSpec((1, _UNITS), lambda i: (0, 0)),
        ],
        out_specs=pl.BlockSpec((_BN, _UNITS), lambda i: (i, 0)),
        out_shape=jax.ShapeDtypeStruct((_N_SC, _UNITS), jnp.float32),
    )(x, med, ws, wn, bias2)


def kernel(x, neigh_x, kernel_self, kernel_neigh, bias):
    bias2 = bias.reshape(1, _UNITS)
    med_lo = _sc_median_kernel()(neigh_x)
    out_lo = _tc_matmul(x, med_lo, kernel_self, kernel_neigh, bias2)
    if _N_TC == 0:
        return out_lo
    out_hi = _tc_fused(x, neigh_x, kernel_self, kernel_neigh, bias2)
    return jnp.concatenate([out_lo, out_hi], axis=0)


# speedup vs baseline: 1.0487x; 1.0487x over previous
import functools

import jax
import jax.numpy as jnp
from jax import lax
from jax.experimental import pallas as pl
from jax.experimental.pallas import tpu as pltpu
from jax.experimental.pallas import tpu_sc as plsc

_N = 10000
_DEG = 32
_D = 128
_UNITS = 128
_LANES = 16
_NC = 2
_NS = 16
_NW = _NC * _NS
_CHUNK = 8

_BN = 1000
_N_SC = 8000
_N_TC = _N - _N_SC


def _oems_pairs(n):
    pairs = []

    def _sort(lo, m):
        if m > 1:
            k = m // 2
            _sort(lo, k)
            _sort(lo + k, k)
            _merge(lo, m, 1)

    def _merge(lo, m, r):
        step = r * 2
        if step < m:
            _merge(lo, m, step)
            _merge(lo + r, m, step)
            for i in range(lo + r, lo + m - r, step):
                pairs.append((i, i + r))
        else:
            pairs.append((lo, lo + r))

    _sort(0, n)
    return pairs


_PAIRS16 = _oems_pairs(16)


def _median32(vals):
    a = list(vals[:16])
    b = list(vals[16:])
    for i, j in _PAIRS16:
        lo = jnp.minimum(a[i], a[j])
        a[j] = jnp.maximum(a[i], a[j])
        a[i] = lo
        lo = jnp.minimum(b[i], b[j])
        b[j] = jnp.maximum(b[i], b[j])
        b[i] = lo
    cand = [jnp.maximum(a[i], b[15 - i]) for i in range(16)]
    while len(cand) > 1:
        cand = [jnp.minimum(cand[2 * i], cand[2 * i + 1])
                for i in range(len(cand) // 2)]
    return cand[0]


_NCHUNKS = _N_SC // _CHUNK
_MAX_NK = (_NCHUNKS + _NW - 1) // _NW
_NPAIRS = (_MAX_NK + 1) // 2


def _sc_median_body(neigh_hbm, med_hbm, buf0, buf1, obuf0, obuf1,
                    isem0, isem1, osem0, osem1):
    wid = lax.axis_index("s") * _NC + lax.axis_index("c")
    nk = (_NCHUNKS - wid + (_NW - 1)) // _NW
    bufs = (buf0, buf1)
    obufs = (obuf0, obuf1)
    isems = (isem0, isem1)
    osems = (osem0, osem1)

    def _in_copy(k, b):
        base = (wid + k * _NW) * _CHUNK
        return pltpu.make_async_copy(
            neigh_hbm.at[pl.ds(base, _CHUNK)], bufs[b], isems[b])

    def _out_copy(k, b):
        base = (wid + k * _NW) * _CHUNK
        return pltpu.make_async_copy(
            obufs[b], med_hbm.at[pl.ds(base, _CHUNK)], osems[b])

    _in_copy(0, 0).start()
    _in_copy(1, 1).start()

    def _compute(b):
        buf = bufs[b]
        obuf = obufs[b]

        def node_body(nd, c):
            for fg in range(_D // _LANES):
                col = fg * _LANES
                vals = [buf[nd, s, pl.ds(col, _LANES)] for s in range(_DEG)]
                obuf[nd, pl.ds(col, _LANES)] = _median32(vals)
            return c

        lax.fori_loop(0, _CHUNK, node_body, 0)

    def pair_body(p, c):
        for b in range(2):
            k = 2 * p + b

            def slot(k=k, b=b):
                _in_copy(k, b).wait()
                pl.when(k >= 2)(lambda: _out_copy(k - 2, b).wait())
                _compute(b)
                _out_copy(k, b).start()
                pl.when(k + 2 < nk)(lambda: _in_copy(k + 2, b).start())

            pl.when(k < nk)(slot)
        return c

    lax.fori_loop(0, _NPAIRS, pair_body, 0)
    _out_copy(nk - 2 + (nk % 2), 0).wait()
    _out_copy(nk - 1 - (nk % 2), 1).wait()


@functools.cache
def _sc_median_kernel():
    return pl.kernel(
        _sc_median_body,
        out_type=jax.ShapeDtypeStruct((_N_SC, _D), jnp.float32),
        mesh=plsc.VectorSubcoreMesh(core_axis_name="c", subcore_axis_name="s",
                                    num_cores=_NC, num_subcores=_NS),
        scratch_types=[
            pltpu.VMEM((_CHUNK, _DEG, _D), jnp.float32),
            pltpu.VMEM((_CHUNK, _DEG, _D), jnp.float32),
            pltpu.VMEM((_CHUNK, _D), jnp.float32),
            pltpu.VMEM((_CHUNK, _D), jnp.float32),
            pltpu.SemaphoreType.DMA,
            pltpu.SemaphoreType.DMA,
            pltpu.SemaphoreType.DMA,
            pltpu.SemaphoreType.DMA,
        ],
    )


_BNT = 16


def _tc_fused_body(x_ref, neigh_ref, ws_ref, wn_ref, b_ref, o_ref, med_ref):
    for r in range(_BNT // 8):
        rows = pl.ds(r * 8, 8)
        med_ref[rows, :] = _median32(
            [neigh_ref[rows, s, :] for s in range(_DEG)])
    out = jnp.dot(x_ref[...], ws_ref[...], preferred_element_type=jnp.float32)
    out = out + jnp.dot(med_ref[...], wn_ref[...],
                        preferred_element_type=jnp.float32)
    o_ref[...] = jnp.maximum(out + b_ref[...], 0.0)


def _tc_fused(x, neigh_x, ws, wn, bias2):
    off = _N_SC // _BNT
    return pl.pallas_call(
        _tc_fused_body,
        grid=(_N_TC // _BNT,),
        in_specs=[
            pl.BlockSpec((_BNT, _D), lambda i: (off + i, 0)),
            pl.BlockSpec((_BNT, _DEG, _D), lambda i: (off + i, 0, 0)),
            pl.BlockSpec((_D, _UNITS), lambda i: (0, 0)),
            pl.BlockSpec((_D, _UNITS), lambda i: (0, 0)),
            pl.BlockSpec((1, _UNITS), lambda i: (0, 0)),
        ],
        out_specs=pl.BlockSpec((_BNT, _UNITS), lambda i: (i, 0)),
        out_shape=jax.ShapeDtypeStruct((_N_TC, _UNITS), jnp.float32),
        scratch_shapes=[pltpu.VMEM((_BNT, _D), jnp.float32)],
    )(x, neigh_x, ws, wn, bias2)


def _tc_finish_body(x_ref, med_ref, hi_ref, ws_ref, wn_ref, b_ref, o_ref):
    i = pl.program_id(0)
    nb_mm = _N_SC // _BN

    @pl.when(i < nb_mm)
    def _mm():
        acc = jnp.dot(x_ref[...], ws_ref[...],
                      preferred_element_type=jnp.float32)
        acc = acc + jnp.dot(med_ref[...], wn_ref[...],
                            preferred_element_type=jnp.float32)
        o_ref[...] = jnp.maximum(acc + b_ref[...], 0.0)

    @pl.when(i >= nb_mm)
    def _copy():
        o_ref[...] = hi_ref[...]


def _tc_finish(x, med, out_hi, ws, wn, bias2):
    nb_mm = _N_SC // _BN
    return pl.pallas_call(
        _tc_finish_body,
        grid=(_N // _BN,),
        in_specs=[
            pl.BlockSpec((_BN, _D), lambda i: (i, 0)),
            pl.BlockSpec((_BN, _D),
                         lambda i: (jnp.minimum(i, _N_SC // _BN - 1), 0)),
            pl.BlockSpec((_BN, _UNITS),
                         lambda i: (jnp.maximum(i - _N_SC // _BN, 0), 0)),
            pl.BlockSpec((_D, _UNITS), lambda i: (0, 0)),
            pl.BlockSpec((_D, _UNITS), lambda i: (0, 0)),
            pl.BlockSpec((1, _UNITS), lambda i: (0, 0)),
        ],
        out_specs=pl.BlockSpec((_BN, _UNITS), lambda i: (i, 0)),
        out_shape=jax.ShapeDtypeStruct((_N, _UNITS), jnp.float32),
    )(x, med, out_hi, ws, wn, bias2)


def kernel(x, neigh_x, kernel_self, kernel_neigh, bias):
    bias2 = bias.reshape(1, _UNITS)
    med_lo = _sc_median_kernel()(neigh_x)
    out_hi = _tc_fused(x, neigh_x, kernel_self, kernel_neigh, bias2)
    return _tc_finish(x, med_lo, out_hi, kernel_self, kernel_neigh, bias2)


# speedup vs baseline: 1.0918x; 1.0411x over previous
import functools

import jax
import jax.numpy as jnp
from jax import lax
from jax.experimental import pallas as pl
from jax.experimental.pallas import tpu as pltpu
from jax.experimental.pallas import tpu_sc as plsc

_N = 10000
_DEG = 32
_D = 128
_UNITS = 128
_LANES = 16
_NC = 2
_NS = 16
_NW = _NC * _NS
_CHUNK = 8

_BN = 1000
_N_SC = 7680
_N_SC_PAD = 8000
_N_TC = _N - _N_SC


def _oems_pairs(n):
    pairs = []

    def _sort(lo, m):
        if m > 1:
            k = m // 2
            _sort(lo, k)
            _sort(lo + k, k)
            _merge(lo, m, 1)

    def _merge(lo, m, r):
        step = r * 2
        if step < m:
            _merge(lo, m, step)
            _merge(lo + r, m, step)
            for i in range(lo + r, lo + m - r, step):
                pairs.append((i, i + r))
        else:
            pairs.append((lo, lo + r))

    _sort(0, n)
    return pairs


_PAIRS16 = _oems_pairs(16)


def _median32(vals):
    a = list(vals[:16])
    b = list(vals[16:])
    for i, j in _PAIRS16:
        lo = jnp.minimum(a[i], a[j])
        a[j] = jnp.maximum(a[i], a[j])
        a[i] = lo
        lo = jnp.minimum(b[i], b[j])
        b[j] = jnp.maximum(b[i], b[j])
        b[i] = lo
    cand = [jnp.maximum(a[i], b[15 - i]) for i in range(16)]
    while len(cand) > 1:
        cand = [jnp.minimum(cand[2 * i], cand[2 * i + 1])
                for i in range(len(cand) // 2)]
    return cand[0]


_NCHUNKS = _N_SC // _CHUNK
_MAX_NK = (_NCHUNKS + _NW - 1) // _NW
_NPAIRS = (_MAX_NK + 1) // 2


def _sc_median_body(neigh_hbm, med_hbm, buf0, buf1, obuf0, obuf1,
                    isem0, isem1, osem0, osem1):
    wid = lax.axis_index("s") * _NC + lax.axis_index("c")
    nk = (_NCHUNKS - wid + (_NW - 1)) // _NW
    bufs = (buf0, buf1)
    obufs = (obuf0, obuf1)
    isems = (isem0, isem1)
    osems = (osem0, osem1)

    def _in_copy(k, b):
        base = (wid + k * _NW) * _CHUNK
        return pltpu.make_async_copy(
            neigh_hbm.at[pl.ds(base, _CHUNK)], bufs[b], isems[b])

    def _out_copy(k, b):
        base = (wid + k * _NW) * _CHUNK
        return pltpu.make_async_copy(
            obufs[b], med_hbm.at[pl.ds(base, _CHUNK)], osems[b])

    _in_copy(0, 0).start()
    _in_copy(1, 1).start()

    def _compute(b):
        buf = bufs[b]
        obuf = obufs[b]

        def node_body(nd, c):
            for fg in range(_D // _LANES):
                col = fg * _LANES
                vals = [buf[nd, s, pl.ds(col, _LANES)] for s in range(_DEG)]
                obuf[nd, pl.ds(col, _LANES)] = _median32(vals)
            return c

        lax.fori_loop(0, _CHUNK, node_body, 0)

    def pair_body(p, c):
        for b in range(2):
            k = 2 * p + b

            def slot(k=k, b=b):
                _in_copy(k, b).wait()
                pl.when(k >= 2)(lambda: _out_copy(k - 2, b).wait())
                _compute(b)
                _out_copy(k, b).start()
                pl.when(k + 2 < nk)(lambda: _in_copy(k + 2, b).start())

            pl.when(k < nk)(slot)
        return c

    lax.fori_loop(0, _NPAIRS, pair_body, 0)
    _out_copy(nk - 2 + (nk % 2), 0).wait()
    _out_copy(nk - 1 - (nk % 2), 1).wait()


@functools.cache
def _sc_median_kernel():
    return pl.kernel(
        _sc_median_body,
        out_type=jax.ShapeDtypeStruct((_N_SC_PAD, _D), jnp.float32),
        mesh=plsc.VectorSubcoreMesh(core_axis_name="c", subcore_axis_name="s",
                                    num_cores=_NC, num_subcores=_NS),
        scratch_types=[
            pltpu.VMEM((_CHUNK, _DEG, _D), jnp.float32),
            pltpu.VMEM((_CHUNK, _DEG, _D), jnp.float32),
            pltpu.VMEM((_CHUNK, _D), jnp.float32),
            pltpu.VMEM((_CHUNK, _D), jnp.float32),
            pltpu.SemaphoreType.DMA,
            pltpu.SemaphoreType.DMA,
            pltpu.SemaphoreType.DMA,
            pltpu.SemaphoreType.DMA,
        ],
    )


_BNT = 40


def _tc_fused_body(x_ref, neigh_ref, ws_ref, wn_ref, b_ref, o_ref, med_ref):
    for r in range(_BNT // 8):
        rows = pl.ds(r * 8, 8)
        med_ref[rows, :] = _median32(
            [neigh_ref[rows, s, :] for s in range(_DEG)])
    out = jnp.dot(x_ref[...], ws_ref[...], preferred_element_type=jnp.float32)
    out = out + jnp.dot(med_ref[...], wn_ref[...],
                        preferred_element_type=jnp.float32)
    o_ref[...] = jnp.maximum(out + b_ref[...], 0.0)


def _tc_fused(x, neigh_x, ws, wn, bias2):
    off = _N_SC // _BNT
    return pl.pallas_call(
        _tc_fused_body,
        grid=(_N_TC // _BNT,),
        in_specs=[
            pl.BlockSpec((_BNT, _D), lambda i: (off + i, 0)),
            pl.BlockSpec((_BNT, _DEG, _D), lambda i: (off + i, 0, 0)),
            pl.BlockSpec((_D, _UNITS), lambda i: (0, 0)),
            pl.BlockSpec((_D, _UNITS), lambda i: (0, 0)),
            pl.BlockSpec((1, _UNITS), lambda i: (0, 0)),
        ],
        out_specs=pl.BlockSpec((_BNT, _UNITS), lambda i: (off + i, 0)),
        out_shape=jax.ShapeDtypeStruct((_N, _UNITS), jnp.float32),
        scratch_shapes=[pltpu.VMEM((_BNT, _D), jnp.float32)],
    )(x, neigh_x, ws, wn, bias2)


_IB = _N_SC // _BN
_IBROW = _N_SC - _IB * _BN


def _tc_finish_body(x_ref, med_ref, hi_ref, ws_ref, wn_ref, b_ref, o_ref):
    i = pl.program_id(0)

    @pl.when(i <= _IB)
    def _mm():
        acc = jnp.dot(x_ref[...], ws_ref[...],
                      preferred_element_type=jnp.float32)
        acc = acc + jnp.dot(med_ref[...], wn_ref[...],
                            preferred_element_type=jnp.float32)
        o_ref[...] = jnp.maximum(acc + b_ref[...], 0.0)

    @pl.when(i == _IB)
    def _boundary():
        o_ref[pl.ds(_IBROW, _BN - _IBROW), :] = \
            hi_ref[pl.ds(_IBROW, _BN - _IBROW), :]

    @pl.when(i > _IB)
    def _copy():
        o_ref[...] = hi_ref[...]


def _tc_finish(x, med, out_hi, ws, wn, bias2):
    return pl.pallas_call(
        _tc_finish_body,
        grid=(_N // _BN,),
        in_specs=[
            pl.BlockSpec((_BN, _D), lambda i: (i, 0)),
            pl.BlockSpec((_BN, _D),
                         lambda i: (jnp.minimum(i, _N_SC_PAD // _BN - 1), 0)),
            pl.BlockSpec((_BN, _UNITS), lambda i: (i, 0)),
            pl.BlockSpec((_D, _UNITS), lambda i: (0, 0)),
            pl.BlockSpec((_D, _UNITS), lambda i: (0, 0)),
            pl.BlockSpec((1, _UNITS), lambda i: (0, 0)),
        ],
        out_specs=pl.BlockSpec((_BN, _UNITS), lambda i: (i, 0)),
        out_shape=jax.ShapeDtypeStruct((_N, _UNITS), jnp.float32),
    )(x, med, out_hi, ws, wn, bias2)


def kernel(x, neigh_x, kernel_self, kernel_neigh, bias):
    bias2 = bias.reshape(1, _UNITS)
    med_lo = _sc_median_kernel()(neigh_x)
    out_hi = _tc_fused(x, neigh_x, kernel_self, kernel_neigh, bias2)
    return _tc_finish(x, med_lo, out_hi, kernel_self, kernel_neigh, bias2)


# speedup vs baseline: 1.1117x; 1.0183x over previous
import functools

import jax
import jax.numpy as jnp
from jax import lax
from jax.experimental import pallas as pl
from jax.experimental.pallas import tpu as pltpu
from jax.experimental.pallas import tpu_sc as plsc

_N = 10000
_DEG = 32
_D = 128
_UNITS = 128
_LANES = 16
_NC = 2
_NS = 16
_NW = _NC * _NS
_CHUNK = 8

_BN = 2000
_N_SC = 7680
_N_SC_PAD = 8000
_N_TC = _N - _N_SC


def _oems_pairs(n):
    pairs = []

    def _sort(lo, m):
        if m > 1:
            k = m // 2
            _sort(lo, k)
            _sort(lo + k, k)
            _merge(lo, m, 1)

    def _merge(lo, m, r):
        step = r * 2
        if step < m:
            _merge(lo, m, step)
            _merge(lo + r, m, step)
            for i in range(lo + r, lo + m - r, step):
                pairs.append((i, i + r))
        else:
            pairs.append((lo, lo + r))

    _sort(0, n)
    return pairs


_PAIRS16 = _oems_pairs(16)


def _median32(vals):
    a = list(vals[:16])
    b = list(vals[16:])
    for i, j in _PAIRS16:
        lo = jnp.minimum(a[i], a[j])
        a[j] = jnp.maximum(a[i], a[j])
        a[i] = lo
        lo = jnp.minimum(b[i], b[j])
        b[j] = jnp.maximum(b[i], b[j])
        b[i] = lo
    cand = [jnp.maximum(a[i], b[15 - i]) for i in range(16)]
    while len(cand) > 1:
        cand = [jnp.minimum(cand[2 * i], cand[2 * i + 1])
                for i in range(len(cand) // 2)]
    return cand[0]


_NCHUNKS = _N_SC // _CHUNK
_MAX_NK = (_NCHUNKS + _NW - 1) // _NW
_NPAIRS = (_MAX_NK + 1) // 2


def _sc_median_body(neigh_hbm, med_hbm, buf0, buf1, obuf0, obuf1,
                    isem0, isem1, osem0, osem1):
    wid = lax.axis_index("s") * _NC + lax.axis_index("c")
    nk = (_NCHUNKS - wid + (_NW - 1)) // _NW
    bufs = (buf0, buf1)
    obufs = (obuf0, obuf1)
    isems = (isem0, isem1)
    osems = (osem0, osem1)

    def _in_copy(k, b):
        base = (wid + k * _NW) * _CHUNK
        return pltpu.make_async_copy(
            neigh_hbm.at[pl.ds(base, _CHUNK)], bufs[b], isems[b])

    def _out_copy(k, b):
        base = (wid + k * _NW) * _CHUNK
        return pltpu.make_async_copy(
            obufs[b], med_hbm.at[pl.ds(base, _CHUNK)], osems[b])

    _in_copy(0, 0).start()
    _in_copy(1, 1).start()

    def _compute(b):
        buf = bufs[b]
        obuf = obufs[b]

        def node_body(nd, c):
            for fg in range(_D // _LANES):
                col = fg * _LANES
                vals = [buf[nd, s, pl.ds(col, _LANES)] for s in range(_DEG)]
                obuf[nd, pl.ds(col, _LANES)] = _median32(vals)
            return c

        lax.fori_loop(0, _CHUNK, node_body, 0)

    def pair_body(p, c):
        for b in range(2):
            k = 2 * p + b

            def slot(k=k, b=b):
                _in_copy(k, b).wait()
                pl.when(k >= 2)(lambda: _out_copy(k - 2, b).wait())
                _compute(b)
                _out_copy(k, b).start()
                pl.when(k + 2 < nk)(lambda: _in_copy(k + 2, b).start())

            pl.when(k < nk)(slot)
        return c

    lax.fori_loop(0, _NPAIRS, pair_body, 0)
    _out_copy(nk - 2 + (nk % 2), 0).wait()
    _out_copy(nk - 1 - (nk % 2), 1).wait()


@functools.cache
def _sc_median_kernel():
    return pl.kernel(
        _sc_median_body,
        out_type=jax.ShapeDtypeStruct((_N_SC_PAD, _D), jnp.float32),
        mesh=plsc.VectorSubcoreMesh(core_axis_name="c", subcore_axis_name="s",
                                    num_cores=_NC, num_subcores=_NS),
        scratch_types=[
            pltpu.VMEM((_CHUNK, _DEG, _D), jnp.float32),
            pltpu.VMEM((_CHUNK, _DEG, _D), jnp.float32),
            pltpu.VMEM((_CHUNK, _D), jnp.float32),
            pltpu.VMEM((_CHUNK, _D), jnp.float32),
            pltpu.SemaphoreType.DMA,
            pltpu.SemaphoreType.DMA,
            pltpu.SemaphoreType.DMA,
            pltpu.SemaphoreType.DMA,
        ],
    )


_BNT = 40


def _tc_fused_body(x_ref, neigh_ref, ws_ref, wn_ref, b_ref, o_ref, med_ref):
    for r in range(_BNT // 8):
        rows = pl.ds(r * 8, 8)
        med_ref[rows, :] = _median32(
            [neigh_ref[rows, s, :] for s in range(_DEG)])
    out = jnp.dot(x_ref[...], ws_ref[...], preferred_element_type=jnp.float32)
    out = out + jnp.dot(med_ref[...], wn_ref[...],
                        preferred_element_type=jnp.float32)
    o_ref[...] = jnp.maximum(out + b_ref[...], 0.0)


def _tc_fused(x, neigh_x, ws, wn, bias2):
    off = _N_SC // _BNT
    return pl.pallas_call(
        _tc_fused_body,
        grid=(_N_TC // _BNT,),
        in_specs=[
            pl.BlockSpec((_BNT, _D), lambda i: (off + i, 0)),
            pl.BlockSpec((_BNT, _DEG, _D), lambda i: (off + i, 0, 0)),
            pl.BlockSpec((_D, _UNITS), lambda i: (0, 0)),
            pl.BlockSpec((_D, _UNITS), lambda i: (0, 0)),
            pl.BlockSpec((1, _UNITS), lambda i: (0, 0)),
        ],
        out_specs=pl.BlockSpec((_BNT, _UNITS), lambda i: (off + i, 0)),
        out_shape=jax.ShapeDtypeStruct((_N, _UNITS), jnp.float32),
        scratch_shapes=[pltpu.VMEM((_BNT, _D), jnp.float32)],
    )(x, neigh_x, ws, wn, bias2)


_IB = _N_SC // _BN
_IBROW = _N_SC - _IB * _BN


def _tc_finish_body(x_ref, med_ref, hi_ref, ws_ref, wn_ref, b_ref, o_ref):
    i = pl.program_id(0)

    @pl.when(i <= _IB)
    def _mm():
        acc = jnp.dot(x_ref[...], ws_ref[...],
                      preferred_element_type=jnp.float32)
        acc = acc + jnp.dot(med_ref[...], wn_ref[...],
                            preferred_element_type=jnp.float32)
        o_ref[...] = jnp.maximum(acc + b_ref[...], 0.0)

    @pl.when(i == _IB)
    def _boundary():
        o_ref[pl.ds(_IBROW, _BN - _IBROW), :] = \
            hi_ref[pl.ds(_IBROW, _BN - _IBROW), :]

    @pl.when(i > _IB)
    def _copy():
        o_ref[...] = hi_ref[...]


def _tc_finish(x, med, out_hi, ws, wn, bias2):
    return pl.pallas_call(
        _tc_finish_body,
        grid=(_N // _BN,),
        in_specs=[
            pl.BlockSpec((_BN, _D), lambda i: (i, 0)),
            pl.BlockSpec((_BN, _D),
                         lambda i: (jnp.minimum(i, _N_SC_PAD // _BN - 1), 0)),
            pl.BlockSpec((_BN, _UNITS), lambda i: (i, 0)),
            pl.BlockSpec((_D, _UNITS), lambda i: (0, 0)),
            pl.BlockSpec((_D, _UNITS), lambda i: (0, 0)),
            pl.BlockSpec((1, _UNITS), lambda i: (0, 0)),
        ],
        out_specs=pl.BlockSpec((_BN, _UNITS), lambda i: (i, 0)),
        out_shape=jax.ShapeDtypeStruct((_N, _UNITS), jnp.float32),
    )(x, med, out_hi, ws, wn, bias2)


def kernel(x, neigh_x, kernel_self, kernel_neigh, bias):
    bias2 = bias.reshape(1, _UNITS)
    med_lo = _sc_median_kernel()(neigh_x)
    out_hi = _tc_fused(x, neigh_x, kernel_self, kernel_neigh, bias2)
    return _tc_finish(x, med_lo, out_hi, kernel_self, kernel_neigh, bias2)
